# Initial kernel scaffold; baseline (speedup 1.0000x reference)
import jax
import jax.numpy as jnp
from jax.experimental import pallas as pl


def kernel(x, orbitals_mf, orbitals_hf):
    raise NotImplementedError("write your pallas kernel here")


# speedup vs baseline: 1.6786x; 1.6786x over previous
import functools

import jax
import jax.numpy as jnp
from jax import lax
from jax.experimental import pallas as pl
from jax.experimental.pallas import tpu as pltpu
from jax.experimental.pallas import tpu_sc as plsc

_N_SAMPLES = 512
_N_SITES = 256
_D = 512
_L = 16
_NC = 2
_NS = 16
_NW = _NC * _NS
_SPW = _N_SAMPLES // _NW
_RCH = 64
_NGC = _N_SITES // _RCH


def _sc_orbitals(x, table):
    mesh = plsc.VectorSubcoreMesh(core_axis_name="c", subcore_axis_name="s")

    @functools.partial(
        pl.kernel,
        out_type=jax.ShapeDtypeStruct((_N_SAMPLES, _N_SITES, _D), jnp.float32),
        mesh=mesh,
        compiler_params=pltpu.CompilerParams(needs_layout_passes=False),
        scratch_types=[
            pltpu.VMEM((_N_SITES,), jnp.int32),
            pltpu.VMEM((_NGC, _RCH), jnp.int32),
            pltpu.VMEM((_RCH, _D), jnp.float32),
            pltpu.VMEM((_RCH, _D), jnp.float32),
            pltpu.SemaphoreType.DMA,
            pltpu.SemaphoreType.DMA,
        ],
    )
    def k(x_hbm, tab_hbm, out_hbm, xv, idxv, rows0, rows1, sem0, sem1):
        wid = lax.axis_index("s") * _NC + lax.axis_index("c")
        bufs = (rows0, rows1)
        sems = (sem0, sem1)

        def do_sample(t, carry):
            s = wid * _SPW + t
            pltpu.sync_copy(x_hbm.at[s], xv)

            one_c = jnp.broadcast_to(jnp.int32(1), (_L,))
            m = jnp.int32(0)
            for c in range(_N_SITES // _L):
                raw = xv[pl.ds(c * _L, _L)]
                m = m + jnp.sum(jnp.where(raw == one_c, one_c, one_c - one_c))

            ones_cum = jnp.int32(0)
            one_v = jnp.broadcast_to(jnp.int32(1), (_L,))
            for c in range(_N_SITES // _L):
                raw = xv[pl.ds(c * _L, _L)]
                occ = jnp.where(raw == one_v, one_v, one_v - one_v)
                cs = lax.cumsum(occ, axis=0)
                zcs = lax.cumsum(one_v - occ, axis=0)
                ones_off = jnp.broadcast_to(ones_cum - 1, (_L,))
                zeros_off = jnp.broadcast_to(m + (c * _L - 1) - ones_cum, (_L,))
                rank = jnp.where(occ == one_v, ones_off + cs, zeros_off + zcs)
                j = lax.iota(jnp.int32, _L) + jnp.broadcast_to(jnp.int32(c * _L), (_L,))
                plsc.store_scatter(
                    idxv,
                    [
                        lax.shift_right_logical(rank, jnp.broadcast_to(jnp.int32(6), (_L,))),
                        jnp.bitwise_and(rank, jnp.broadcast_to(jnp.int32(63), (_L,))),
                    ],
                    j,
                )
                ones_cum = ones_cum + jnp.sum(occ)

            cps = [None, None]
            for g in range(_NGC):
                b = g % 2
                if cps[b] is not None:
                    cps[b].wait()
                    pltpu.sync_copy(
                        bufs[b], out_hbm.at[s, pl.ds((g - 2) * _RCH, _RCH)]
                    )
                cps[b] = pltpu.async_copy(tab_hbm.at[idxv.at[g]], bufs[b], sems[b])
            for g in range(_NGC - 2, _NGC):
                b = g % 2
                cps[b].wait()
                pltpu.sync_copy(bufs[b], out_hbm.at[s, pl.ds(g * _RCH, _RCH)])
            return carry

        lax.fori_loop(0, _SPW, do_sample, jnp.int32(0))

    return k(x, table)


def kernel(x, orbitals_mf, orbitals_hf):
    table = jnp.concatenate([orbitals_mf, orbitals_hf], axis=1)
    return _sc_orbitals(x, table)
